# Initial kernel scaffold; baseline (speedup 1.0000x reference)
import jax
import jax.numpy as jnp
from jax.experimental import pallas as pl


def kernel(X, edge_index, W1, W2):
    raise NotImplementedError("write your pallas kernel here")


# speedup vs baseline: 39.6366x; 39.6366x over previous
import functools

import jax
import jax.numpy as jnp
from jax import lax
from jax.experimental import pallas as pl
from jax.experimental.pallas import tpu as pltpu
from jax.experimental.pallas import tpu_sc as plsc

N_NODES = 10000
N_EDGES = 320000
NC = 2
NS = 16
NW = NC * NS
E_PER_W = N_EDGES // NW
LANES = 16
N_STEPS = E_PER_W // LANES


def _matmul_body(x_ref, w_ref, y_ref):
    y_ref[...] = lax.dot_general(
        w_ref[...], x_ref[...],
        dimension_numbers=(((0,), (1,)), ((), ())),
        preferred_element_type=jnp.float32,
    )


def _node_tables(X, W1, W2):
    W = jnp.concatenate([W1, W2], axis=1)
    return pl.pallas_call(
        _matmul_body,
        out_shape=jax.ShapeDtypeStruct((2, N_NODES), jnp.float32),
    )(X, W)


def _edge_body(y_hbm, ei_hbm, out_hbm, y1_v, y2_v, src_v, dst_v, out_v):
    wid = lax.axis_index("s") * NC + lax.axis_index("c")
    base = wid * E_PER_W
    pltpu.sync_copy(y_hbm.at[pl.ds(0, N_NODES)], y1_v)
    pltpu.sync_copy(y_hbm.at[pl.ds(N_NODES, N_NODES)], y2_v)
    pltpu.sync_copy(ei_hbm.at[pl.ds(base, E_PER_W)], src_v)
    pltpu.sync_copy(ei_hbm.at[pl.ds(N_EDGES + base, E_PER_W)], dst_v)

    @plsc.parallel_loop(0, E_PER_W, LANES, unroll=8)
    def _(off):
        s_idx = src_v[pl.ds(off, LANES)]
        d_idx = dst_v[pl.ds(off, LANES)]
        g1 = plsc.load_gather(y1_v, [s_idx])
        g2 = plsc.load_gather(y2_v, [d_idx])
        out_v[pl.ds(off, LANES)] = g1 + g2
    pltpu.sync_copy(out_v, out_hbm.at[pl.ds(base, E_PER_W)])


_edge_call = pl.kernel(
    _edge_body,
    out_type=jax.ShapeDtypeStruct((N_EDGES,), jnp.float32),
    mesh=plsc.VectorSubcoreMesh(core_axis_name="c", subcore_axis_name="s"),
    compiler_params=pltpu.CompilerParams(needs_layout_passes=False),
    scratch_types=[
        pltpu.VMEM((N_NODES,), jnp.float32),
        pltpu.VMEM((N_NODES,), jnp.float32),
        pltpu.VMEM((E_PER_W,), jnp.int32),
        pltpu.VMEM((E_PER_W,), jnp.int32),
        pltpu.VMEM((E_PER_W,), jnp.float32),
    ],
)


def kernel(X, edge_index, W1, W2):
    Y = _node_tables(X, W1, W2)
    ei = edge_index.astype(jnp.int32).reshape(-1)
    out = _edge_call(Y.reshape(-1), ei)
    return out.reshape(N_EDGES, 1)


# speedup vs baseline: 46.1925x; 1.1654x over previous
import jax
import jax.numpy as jnp
from jax import lax
from jax.experimental import pallas as pl
from jax.experimental.pallas import tpu as pltpu
from jax.experimental.pallas import tpu_sc as plsc

N_NODES = 10000
N_EDGES = 320000
NC = 2
NS = 16
NW = NC * NS
LANES = 16
TILE = 128
N_TILES = N_EDGES // TILE
E_MAX = (N_TILES // NW + 1) * TILE


def _matmul_body(x_ref, w1_ref, w2_ref, y_ref):
    w = jnp.concatenate([w1_ref[...], w2_ref[...]], axis=1)
    y_ref[...] = lax.dot_general(
        w, x_ref[...],
        dimension_numbers=(((0,), (1,)), ((), ())),
        preferred_element_type=jnp.float32,
    )


def _node_tables(X, W1, W2):
    return pl.pallas_call(
        _matmul_body,
        out_shape=jax.ShapeDtypeStruct((2, N_NODES), jnp.float32),
    )(X, W1, W2)


def _edge_body(y_hbm, ei_hbm, out_hbm, y_v, ei_v, out_v, sem_y, sem_ei):
    wid = lax.axis_index("s") * NC + lax.axis_index("c")
    t0 = wid * N_TILES // NW
    t1 = (wid + 1) * N_TILES // NW
    base = t0 * TILE
    n_w = (t1 - t0) * TILE

    cp_y = pltpu.async_copy(y_hbm, y_v, sem_y)
    cp_ei = pltpu.async_copy(ei_hbm.at[:, pl.ds(base, n_w)],
                             ei_v.at[:, pl.ds(0, n_w)], sem_ei)
    cp_y.wait()
    cp_ei.wait()

    zero = jnp.zeros((LANES,), jnp.int32)
    one = zero + 1

    @plsc.parallel_loop(0, n_w, LANES, unroll=8)
    def _(off):
        s_idx = ei_v[0, pl.ds(off, LANES)]
        d_idx = ei_v[1, pl.ds(off, LANES)]
        g1 = plsc.load_gather(y_v, [zero, s_idx])
        g2 = plsc.load_gather(y_v, [one, d_idx])
        out_v[pl.ds(off, LANES)] = g1 + g2

    pltpu.sync_copy(out_v.at[pl.ds(0, n_w)], out_hbm.at[pl.ds(base, n_w)])


_edge_call = pl.kernel(
    _edge_body,
    out_type=jax.ShapeDtypeStruct((N_EDGES,), jnp.float32),
    mesh=plsc.VectorSubcoreMesh(core_axis_name="c", subcore_axis_name="s"),
    compiler_params=pltpu.CompilerParams(needs_layout_passes=False),
    scratch_types=[
        pltpu.VMEM((2, N_NODES), jnp.float32),
        pltpu.VMEM((2, E_MAX), jnp.int32),
        pltpu.VMEM((E_MAX,), jnp.float32),
        pltpu.SemaphoreType.DMA,
        pltpu.SemaphoreType.DMA,
    ],
)


def kernel(X, edge_index, W1, W2):
    Y = _node_tables(X, W1, W2)
    out = _edge_call(Y, edge_index.astype(jnp.int32))
    return out.reshape(N_EDGES, 1)


# speedup vs baseline: 64.5069x; 1.3965x over previous
import jax
import jax.numpy as jnp
from jax import lax
from jax.experimental import pallas as pl
from jax.experimental.pallas import tpu as pltpu
from jax.experimental.pallas import tpu_sc as plsc

N_NODES = 10000
N_EDGES = 320000
NC = 2
NS = 16
NW = NC * NS
LANES = 16
TILE = 128
N_TILES = N_EDGES // TILE
E_MAX = (N_TILES // NW + 1) * TILE


def _matmul_body(x_ref, w1t_ref, w2t_ref, y_ref):
    wt = jnp.concatenate([w1t_ref[...], w2t_ref[...]], axis=0)
    y_ref[...] = lax.dot_general(
        wt, x_ref[...],
        dimension_numbers=(((1,), (1,)), ((), ())),
        preferred_element_type=jnp.float32,
    )


def _node_tables(X, W1, W2):
    return pl.pallas_call(
        _matmul_body,
        out_shape=jax.ShapeDtypeStruct((2, N_NODES), jnp.float32),
    )(X, W1.T, W2.T)


def _edge_body(y_hbm, ei_hbm, out_hbm, y_v, ei_v, out_v, sem_y, sem_ei):
    wid = lax.axis_index("s") * NC + lax.axis_index("c")
    t0 = wid * N_TILES // NW
    t1 = (wid + 1) * N_TILES // NW
    base = t0 * TILE
    n_w = (t1 - t0) * TILE

    cp_y = pltpu.async_copy(y_hbm, y_v, sem_y)
    cp_ei = pltpu.async_copy(ei_hbm.at[:, pl.ds(base, n_w)],
                             ei_v.at[:, pl.ds(0, n_w)], sem_ei)
    cp_y.wait()
    cp_ei.wait()

    zero = jnp.zeros((LANES,), jnp.int32)
    one = zero + 1

    @plsc.parallel_loop(0, n_w, LANES, unroll=8)
    def _(off):
        s_idx = ei_v[0, pl.ds(off, LANES)]
        d_idx = ei_v[1, pl.ds(off, LANES)]
        g1 = plsc.load_gather(y_v, [zero, s_idx])
        g2 = plsc.load_gather(y_v, [one, d_idx])
        out_v[pl.ds(off, LANES)] = g1 + g2

    pltpu.sync_copy(out_v.at[pl.ds(0, n_w)], out_hbm.at[0, pl.ds(base, n_w)])


_edge_call = pl.kernel(
    _edge_body,
    out_type=jax.ShapeDtypeStruct((1, N_EDGES), jnp.float32),
    mesh=plsc.VectorSubcoreMesh(core_axis_name="c", subcore_axis_name="s"),
    compiler_params=pltpu.CompilerParams(needs_layout_passes=False),
    scratch_types=[
        pltpu.VMEM((2, N_NODES), jnp.float32),
        pltpu.VMEM((2, E_MAX), jnp.int32),
        pltpu.VMEM((E_MAX,), jnp.float32),
        pltpu.SemaphoreType.DMA,
        pltpu.SemaphoreType.DMA,
    ],
)


def kernel(X, edge_index, W1, W2):
    Y = _node_tables(X, W1, W2)
    out = _edge_call(Y, edge_index.astype(jnp.int32))
    return out.reshape(N_EDGES, 1)
